# Initial kernel scaffold; baseline (speedup 1.0000x reference)
import jax
import jax.numpy as jnp
from jax import lax
from jax.experimental import pallas as pl
from jax.experimental.pallas import tpu as pltpu
from jax.experimental.pallas import tpu_sc as plsc

N = 10000
E = 320000
D = 128

NC = 2
NS = 16
NW = NC * NS
EPW = E // NW
CHUNK = 125
NCHUNKS = EPW // CHUNK
RPW = N // NS
RCH = RPW // CHUNK


def _sc_body(x_hbm, eidx_hbm, psum_hbm, pcnt_hbm,
             src_i, dst_i, rows, ones, sem):
    c = lax.axis_index("c")
    s = lax.axis_index("s")
    wid = c * NS + s

    def scoped(summed_sh, cnt_sh):
        @pl.loop(0, CHUNK)
        def _(i):
            for j in range(D // 16):
                rows[i, pl.ds(j * 16, 16)] = jnp.zeros((16,), jnp.float32)

        @pl.loop(0, CHUNK)
        def _(i):
            ones[i, :] = jnp.zeros((16,), jnp.float32)

        base = s * RPW
        for t in range(RCH):
            pltpu.sync_copy(rows, summed_sh.at[pl.ds(base + t * CHUNK, CHUNK)])
            pltpu.sync_copy(ones, cnt_sh.at[pl.ds(base + t * CHUNK, CHUNK)])

        onehot = jnp.where(lax.iota(jnp.int32, 16) == 0,
                           jnp.float32(1.0), jnp.float32(0.0))

        @pl.loop(0, CHUNK)
        def _(i):
            ones[i, :] = onehot

        plsc.subcore_barrier()

        pltpu.sync_copy(eidx_hbm.at[0, wid], src_i)
        pltpu.sync_copy(eidx_hbm.at[1, wid], dst_i)

        @pl.loop(0, NCHUNKS)
        def _(j):
            pltpu.async_copy(x_hbm.at[src_i.at[j]], rows, sem).wait()
            pltpu.sync_copy(rows, summed_sh.at[dst_i.at[j]], add=True)
            pltpu.sync_copy(ones, cnt_sh.at[dst_i.at[j]], add=True)

        plsc.subcore_barrier()

        for t in range(RCH):
            sl = pl.ds(base + t * CHUNK, CHUNK)
            pltpu.sync_copy(summed_sh.at[sl], rows)
            pltpu.sync_copy(rows, psum_hbm.at[c, sl])
            pltpu.sync_copy(cnt_sh.at[sl], ones)
            pltpu.sync_copy(ones, pcnt_hbm.at[c, sl])

    pl.run_scoped(
        scoped,
        pltpu.VMEM_SHARED((N, D), jnp.float32),
        pltpu.VMEM_SHARED((N, 16), jnp.float32),
    )


_sc_aggregate = pl.kernel(
    _sc_body,
    out_type=[
        jax.ShapeDtypeStruct((NC, N, D), jnp.float32),
        jax.ShapeDtypeStruct((NC, N, 16), jnp.float32),
    ],
    mesh=plsc.VectorSubcoreMesh(core_axis_name="c", subcore_axis_name="s"),
    scratch_types=[
        pltpu.VMEM((NCHUNKS, CHUNK), jnp.int32),
        pltpu.VMEM((NCHUNKS, CHUNK), jnp.int32),
        pltpu.VMEM((CHUNK, D), jnp.float32),
        pltpu.VMEM((CHUNK, 16), jnp.float32),
        pltpu.SemaphoreType.DMA,
    ],
)


BN = 1000


def _tc_body(psum_ref, pcnt_ref, x_ref, wl_ref, bl_ref, wr_ref, out_ref):
    ssum = psum_ref[0] + psum_ref[1]
    cnt = pcnt_ref[0][:, 0:1] + pcnt_ref[1][:, 0:1]
    mean = ssum / jnp.maximum(cnt, 1.0)
    out = (jnp.dot(mean, wl_ref[:], preferred_element_type=jnp.float32)
           + bl_ref[:]
           + jnp.dot(x_ref[:], wr_ref[:], preferred_element_type=jnp.float32))
    nrm = jnp.sqrt(jnp.sum(out * out, axis=1, keepdims=True))
    out_ref[:] = out / jnp.maximum(nrm, 1e-12)


def _tc_dense(psum, pcnt, x, W_l, b_l, W_r):
    grid = (N // BN,)
    return pl.pallas_call(
        _tc_body,
        grid=grid,
        in_specs=[
            pl.BlockSpec((NC, BN, D), lambda i: (0, i, 0)),
            pl.BlockSpec((NC, BN, 16), lambda i: (0, i, 0)),
            pl.BlockSpec((BN, D), lambda i: (i, 0)),
            pl.BlockSpec((D, D), lambda i: (0, 0)),
            pl.BlockSpec((1, D), lambda i: (0, 0)),
            pl.BlockSpec((D, D), lambda i: (0, 0)),
        ],
        out_specs=pl.BlockSpec((BN, D), lambda i: (i, 0)),
        out_shape=jax.ShapeDtypeStruct((N, D), jnp.float32),
    )(psum, pcnt, x, W_l, b_l, W_r)


def kernel(x, edge_index, W_l, b_l, W_r):
    eidx = edge_index.reshape(2, NW, NCHUNKS, CHUNK)
    psum, pcnt = _sc_aggregate(x, eidx)
    return _tc_dense(psum, pcnt, x, W_l, b_l.reshape(1, D), W_r)


# speedup vs baseline: 8.0979x; 8.0979x over previous
import jax
import jax.numpy as jnp
from jax import lax
from jax.experimental import pallas as pl
from jax.experimental.pallas import tpu as pltpu
from jax.experimental.pallas import tpu_sc as plsc

N = 10000
E = 320000
D = 128

NC = 2
NS = 16
NW = NC * NS
EPW = E // NW
CHUNK = 125
NCHUNKS = EPW // CHUNK
WB = 80
NWB = N // WB


def _sum_body(x_hbm, eidx_hbm, psum_hbm, src_i, dst_i, rows, summed_sh, sem):
    c = lax.axis_index("c")
    s = lax.axis_index("s")
    wid = c * NS + s

    @pl.loop(0, CHUNK)
    def _(i):
        for j in range(D // 16):
            rows[i, pl.ds(j * 16, 16)] = jnp.zeros((16,), jnp.float32)

    @pl.loop(s, NWB, step=NS)
    def _(t):
        pltpu.sync_copy(rows.at[pl.ds(0, WB)],
                        summed_sh.at[pl.ds(t * WB, WB)])

    plsc.subcore_barrier()

    pltpu.sync_copy(eidx_hbm.at[0, wid], src_i)
    pltpu.sync_copy(eidx_hbm.at[1, wid], dst_i)

    @pl.loop(0, NCHUNKS)
    def _(j):
        pltpu.async_copy(x_hbm.at[src_i.at[j]], rows, sem).wait()
        pltpu.sync_copy(rows, summed_sh.at[dst_i.at[j]], add=True)

    plsc.subcore_barrier()

    @pl.loop(s, NWB, step=NS)
    def _(t):
        sl = pl.ds(t * WB, WB)
        pltpu.sync_copy(summed_sh.at[sl], rows.at[pl.ds(0, WB)])
        pltpu.sync_copy(rows.at[pl.ds(0, WB)], psum_hbm.at[c, sl])


_sc_sum = pl.kernel(
    _sum_body,
    out_type=jax.ShapeDtypeStruct((NC, N, D), jnp.float32),
    mesh=plsc.VectorSubcoreMesh(core_axis_name="c", subcore_axis_name="s"),
    scratch_types=[
        pltpu.VMEM((NCHUNKS, CHUNK), jnp.int32),
        pltpu.VMEM((NCHUNKS, CHUNK), jnp.int32),
        pltpu.VMEM((CHUNK, D), jnp.float32),
        pltpu.VMEM_SHARED((N, D), jnp.float32),
        pltpu.SemaphoreType.DMA,
    ],
)


def _cnt_body(eidx_hbm, pcnt_hbm, dst_i, ones, cnt_sh):
    c = lax.axis_index("c")
    s = lax.axis_index("s")
    wid = c * NS + s

    @pl.loop(0, CHUNK)
    def _(i):
        for j in range(D // 16):
            ones[i, pl.ds(j * 16, 16)] = jnp.zeros((16,), jnp.float32)

    @pl.loop(s, NWB, step=NS)
    def _(t):
        pltpu.sync_copy(ones.at[pl.ds(0, WB)],
                        cnt_sh.at[pl.ds(t * WB, WB)])

    @pl.loop(0, CHUNK)
    def _(i):
        for j in range(D // 16):
            ones[i, pl.ds(j * 16, 16)] = jnp.full((16,), 1.0, jnp.float32)

    plsc.subcore_barrier()

    pltpu.sync_copy(eidx_hbm.at[1, wid], dst_i)

    @pl.loop(0, NCHUNKS)
    def _(j):
        pltpu.sync_copy(ones, cnt_sh.at[dst_i.at[j]], add=True)

    plsc.subcore_barrier()

    @pl.loop(s, NWB, step=NS)
    def _(t):
        sl = pl.ds(t * WB, WB)
        pltpu.sync_copy(cnt_sh.at[sl], ones.at[pl.ds(0, WB)])
        pltpu.sync_copy(ones.at[pl.ds(0, WB)], pcnt_hbm.at[c, sl])


_sc_cnt = pl.kernel(
    _cnt_body,
    out_type=jax.ShapeDtypeStruct((NC, N, D), jnp.float32),
    mesh=plsc.VectorSubcoreMesh(core_axis_name="c", subcore_axis_name="s"),
    scratch_types=[
        pltpu.VMEM((NCHUNKS, CHUNK), jnp.int32),
        pltpu.VMEM((CHUNK, D), jnp.float32),
        pltpu.VMEM_SHARED((N, D), jnp.float32),
    ],
)


BN = 1000


def _tc_body(psum_ref, pcnt_ref, x_ref, wl_ref, bl_ref, wr_ref, out_ref):
    ssum = psum_ref[0] + psum_ref[1]
    cnt = pcnt_ref[0][:, 0:1] + pcnt_ref[1][:, 0:1]
    mean = ssum / jnp.maximum(cnt, 1.0)
    out = (jnp.dot(mean, wl_ref[:], preferred_element_type=jnp.float32)
           + bl_ref[:]
           + jnp.dot(x_ref[:], wr_ref[:], preferred_element_type=jnp.float32))
    nrm = jnp.sqrt(jnp.sum(out * out, axis=1, keepdims=True))
    out_ref[:] = out / jnp.maximum(nrm, 1e-12)


def _tc_dense(psum, pcnt, x, W_l, b_l, W_r):
    return pl.pallas_call(
        _tc_body,
        grid=(N // BN,),
        in_specs=[
            pl.BlockSpec((NC, BN, D), lambda i: (0, i, 0)),
            pl.BlockSpec((NC, BN, D), lambda i: (0, i, 0)),
            pl.BlockSpec((BN, D), lambda i: (i, 0)),
            pl.BlockSpec((D, D), lambda i: (0, 0)),
            pl.BlockSpec((1, D), lambda i: (0, 0)),
            pl.BlockSpec((D, D), lambda i: (0, 0)),
        ],
        out_specs=pl.BlockSpec((BN, D), lambda i: (i, 0)),
        out_shape=jax.ShapeDtypeStruct((N, D), jnp.float32),
    )(psum, pcnt, x, W_l, b_l, W_r)


def kernel(x, edge_index, W_l, b_l, W_r):
    eidx = edge_index.reshape(2, NW, NCHUNKS, CHUNK)
    psum = _sc_sum(x, eidx)
    pcnt = _sc_cnt(eidx)
    return _tc_dense(psum, pcnt, x, W_l, b_l.reshape(1, D), W_r)


# speedup vs baseline: 9.6232x; 1.1884x over previous
import jax
import jax.numpy as jnp
from jax import lax
from jax.experimental import pallas as pl
from jax.experimental.pallas import tpu as pltpu
from jax.experimental.pallas import tpu_sc as plsc

N = 10000
E = 320000
D = 128

NC = 2
NS = 16
NW = NC * NS
EPW = E // NW
CHUNK = 125
NCHUNKS = EPW // CHUNK
WB = 80
NWB = N // WB
NH = 2
HC = NCHUNKS // NH


def _sum_body(x_hbm, eidx_hbm, psum_hbm, src_i, dst_i, rows_a, rows_b,
              summed_sh, sg_a, sg_b):
    c = lax.axis_index("c")
    s = lax.axis_index("s")
    wid = c * NS + s

    @pl.loop(0, CHUNK)
    def _(i):
        for j in range(D // 16):
            rows_a[i, pl.ds(j * 16, 16)] = jnp.zeros((16,), jnp.float32)

    @pl.loop(s, NWB, step=NS)
    def _(t):
        pltpu.sync_copy(rows_a.at[pl.ds(0, WB)],
                        summed_sh.at[pl.ds(t * WB, WB)])

    plsc.subcore_barrier()

    pltpu.sync_copy(eidx_hbm.at[1, wid], dst_i)

    def gather(j, buf, sem):
        pltpu.async_copy(x_hbm.at[src_i.at[j]], buf, sem)

    def gwait(j, buf, sem):
        pltpu.make_async_copy(x_hbm.at[src_i.at[j]], buf, sem).wait()

    for h in range(NH):
        d0 = h * HC
        pltpu.sync_copy(eidx_hbm.at[0, wid, pl.ds(d0, HC)], src_i)
        gather(0, rows_a, sg_a)

        @pl.loop(0, HC // 2 - 1)
        def _(j2):
            j = j2 * 2
            gwait(j, rows_a, sg_a)
            gather(j + 1, rows_b, sg_b)
            pltpu.sync_copy(rows_a, summed_sh.at[dst_i.at[d0 + j]],
                            add=True)
            gwait(j + 1, rows_b, sg_b)
            gather(j + 2, rows_a, sg_a)
            pltpu.sync_copy(rows_b, summed_sh.at[dst_i.at[d0 + j + 1]],
                            add=True)

        jl = HC - 2
        gwait(jl, rows_a, sg_a)
        gather(jl + 1, rows_b, sg_b)
        pltpu.sync_copy(rows_a, summed_sh.at[dst_i.at[d0 + jl]], add=True)
        gwait(jl + 1, rows_b, sg_b)
        pltpu.sync_copy(rows_b, summed_sh.at[dst_i.at[d0 + jl + 1]], add=True)

    plsc.subcore_barrier()

    @pl.loop(s, NWB, step=NS)
    def _(t):
        sl = pl.ds(t * WB, WB)
        pltpu.sync_copy(summed_sh.at[sl], rows_a.at[pl.ds(0, WB)])
        pltpu.sync_copy(rows_a.at[pl.ds(0, WB)], psum_hbm.at[c, sl])


_sc_sum = pl.kernel(
    _sum_body,
    out_type=jax.ShapeDtypeStruct((NC, N, D), jnp.float32),
    mesh=plsc.VectorSubcoreMesh(core_axis_name="c", subcore_axis_name="s"),
    scratch_types=[
        pltpu.VMEM((HC, CHUNK), jnp.int32),
        pltpu.VMEM((NCHUNKS, CHUNK), jnp.int32),
        pltpu.VMEM((CHUNK, D), jnp.float32),
        pltpu.VMEM((CHUNK, D), jnp.float32),
        pltpu.VMEM_SHARED((N, D), jnp.float32),
        pltpu.SemaphoreType.DMA,
        pltpu.SemaphoreType.DMA,
    ],
)


def _cnt_body(eidx_hbm, pcnt_hbm, dst_i, ones, cnt_sh):
    c = lax.axis_index("c")
    s = lax.axis_index("s")
    wid = c * NS + s

    @pl.loop(0, CHUNK)
    def _(i):
        for j in range(D // 16):
            ones[i, pl.ds(j * 16, 16)] = jnp.zeros((16,), jnp.float32)

    @pl.loop(s, NWB, step=NS)
    def _(t):
        pltpu.sync_copy(ones.at[pl.ds(0, WB)],
                        cnt_sh.at[pl.ds(t * WB, WB)])

    @pl.loop(0, CHUNK)
    def _(i):
        for j in range(D // 16):
            ones[i, pl.ds(j * 16, 16)] = jnp.full((16,), 1.0, jnp.float32)

    plsc.subcore_barrier()

    pltpu.sync_copy(eidx_hbm.at[1, wid], dst_i)

    @pl.loop(0, NCHUNKS)
    def _(j):
        pltpu.sync_copy(ones, cnt_sh.at[dst_i.at[j]], add=True)

    plsc.subcore_barrier()

    @pl.loop(s, NWB, step=NS)
    def _(t):
        sl = pl.ds(t * WB, WB)
        pltpu.sync_copy(cnt_sh.at[sl], ones.at[pl.ds(0, WB)])
        pltpu.sync_copy(ones.at[pl.ds(0, WB)], pcnt_hbm.at[c, sl])


_sc_cnt = pl.kernel(
    _cnt_body,
    out_type=jax.ShapeDtypeStruct((NC, N, D), jnp.float32),
    mesh=plsc.VectorSubcoreMesh(core_axis_name="c", subcore_axis_name="s"),
    scratch_types=[
        pltpu.VMEM((NCHUNKS, CHUNK), jnp.int32),
        pltpu.VMEM((CHUNK, D), jnp.float32),
        pltpu.VMEM_SHARED((N, D), jnp.float32),
    ],
)


BN = 1000


def _tc_body(psum_ref, pcnt_ref, x_ref, wl_ref, bl_ref, wr_ref, out_ref):
    ssum = psum_ref[0] + psum_ref[1]
    cnt = pcnt_ref[0][:, 0:1] + pcnt_ref[1][:, 0:1]
    mean = ssum / jnp.maximum(cnt, 1.0)
    out = (jnp.dot(mean, wl_ref[:], preferred_element_type=jnp.float32)
           + bl_ref[:]
           + jnp.dot(x_ref[:], wr_ref[:], preferred_element_type=jnp.float32))
    nrm = jnp.sqrt(jnp.sum(out * out, axis=1, keepdims=True))
    out_ref[:] = out / jnp.maximum(nrm, 1e-12)


def _tc_dense(psum, pcnt, x, W_l, b_l, W_r):
    return pl.pallas_call(
        _tc_body,
        grid=(N // BN,),
        in_specs=[
            pl.BlockSpec((NC, BN, D), lambda i: (0, i, 0)),
            pl.BlockSpec((NC, BN, D), lambda i: (0, i, 0)),
            pl.BlockSpec((BN, D), lambda i: (i, 0)),
            pl.BlockSpec((D, D), lambda i: (0, 0)),
            pl.BlockSpec((1, D), lambda i: (0, 0)),
            pl.BlockSpec((D, D), lambda i: (0, 0)),
        ],
        out_specs=pl.BlockSpec((BN, D), lambda i: (i, 0)),
        out_shape=jax.ShapeDtypeStruct((N, D), jnp.float32),
    )(psum, pcnt, x, W_l, b_l, W_r)


def kernel(x, edge_index, W_l, b_l, W_r):
    eidx = edge_index.reshape(2, NW, NCHUNKS, CHUNK)
    psum = _sc_sum(x, eidx)
    pcnt = _sc_cnt(eidx)
    return _tc_dense(psum, pcnt, x, W_l, b_l.reshape(1, D), W_r)


# speedup vs baseline: 10.8521x; 1.1277x over previous
import jax
import jax.numpy as jnp
from jax import lax
from jax.experimental import pallas as pl
from jax.experimental.pallas import tpu as pltpu
from jax.experimental.pallas import tpu_sc as plsc

N = 10000
E = 320000
D = 128

NC = 2
NS = 16
NW = NC * NS
EPW = E // NW
CHUNK = 125
NCHUNKS = EPW // CHUNK
WB = 80
NWB = N // WB
NH = 2
HC = NCHUNKS // NH


def _fill(buf, value):
    @pl.loop(0, CHUNK)
    def _(i):
        for j in range(D // 16):
            buf[i, pl.ds(j * 16, 16)] = jnp.full((16,), value, jnp.float32)


def _agg_body(x_hbm, eidx_hbm, psum_hbm, pcnt_hbm,
              src_i, dst_i, rows_a, rows_b, acc_sh, sg_a, sg_b):
    c = lax.axis_index("c")
    s = lax.axis_index("s")
    wid = c * NS + s

    _fill(rows_b, 0.0)

    @pl.loop(s, NWB, step=NS)
    def _(t):
        pltpu.sync_copy(rows_b.at[pl.ds(0, WB)],
                        acc_sh.at[pl.ds(t * WB, WB)])

    plsc.subcore_barrier()

    pltpu.sync_copy(eidx_hbm.at[1, wid], dst_i)

    def gather(j, buf, sem):
        pltpu.async_copy(x_hbm.at[src_i.at[j]], buf, sem)

    def gwait(j, buf, sem):
        pltpu.make_async_copy(x_hbm.at[src_i.at[j]], buf, sem).wait()

    bufs = [(rows_a, sg_a), (rows_b, sg_b)]
    NB = len(bufs)

    for h in range(NH):
        d0 = h * HC
        pltpu.sync_copy(eidx_hbm.at[0, wid, pl.ds(d0, HC)], src_i)
        for k in range(NB):
            gather(k, *bufs[k])

        @pl.loop(0, HC // NB - 1)
        def _(jg):
            j = jg * NB
            for k in range(NB):
                buf, sem = bufs[k]
                gwait(j + k, buf, sem)
                pltpu.sync_copy(buf, acc_sh.at[dst_i.at[d0 + j + k]],
                                add=True)
                gather(j + NB + k, buf, sem)

        jl = HC - NB
        for k in range(NB):
            buf, sem = bufs[k]
            gwait(jl + k, buf, sem)
            pltpu.sync_copy(buf, acc_sh.at[dst_i.at[d0 + jl + k]],
                            add=True)

    plsc.subcore_barrier()

    _fill(rows_b, 0.0)

    @pl.loop(s, NWB, step=NS)
    def _(t):
        sl = pl.ds(t * WB, WB)
        pltpu.sync_copy(acc_sh.at[sl], rows_a.at[pl.ds(0, WB)])
        pltpu.sync_copy(rows_a.at[pl.ds(0, WB)], psum_hbm.at[c, sl])
        pltpu.sync_copy(rows_b.at[pl.ds(0, WB)], acc_sh.at[sl])

    plsc.subcore_barrier()

    _fill(rows_a, 1.0)

    @pl.loop(0, NCHUNKS)
    def _(j):
        pltpu.sync_copy(rows_a, acc_sh.at[dst_i.at[j]], add=True)

    plsc.subcore_barrier()

    @pl.loop(s, NWB, step=NS)
    def _(t):
        sl = pl.ds(t * WB, WB)
        pltpu.sync_copy(acc_sh.at[sl], rows_b.at[pl.ds(0, WB)])
        pltpu.sync_copy(rows_b.at[pl.ds(0, WB)], pcnt_hbm.at[c, sl])


_sc_agg = pl.kernel(
    _agg_body,
    out_type=[
        jax.ShapeDtypeStruct((NC, N, D), jnp.float32),
        jax.ShapeDtypeStruct((NC, N, D), jnp.float32),
    ],
    mesh=plsc.VectorSubcoreMesh(core_axis_name="c", subcore_axis_name="s"),
    scratch_types=[
        pltpu.VMEM((HC, CHUNK), jnp.int32),
        pltpu.VMEM((NCHUNKS, CHUNK), jnp.int32),
        pltpu.VMEM((CHUNK, D), jnp.float32),
        pltpu.VMEM((CHUNK, D), jnp.float32),
        pltpu.VMEM_SHARED((N, D), jnp.float32),
        pltpu.SemaphoreType.DMA,
        pltpu.SemaphoreType.DMA,
    ],
)


BN = 1000


def _tc_body(psum_ref, pcnt_ref, x_ref, wl_ref, bl_ref, wr_ref, out_ref):
    ssum = psum_ref[0] + psum_ref[1]
    cnt = pcnt_ref[0][:, 0:1] + pcnt_ref[1][:, 0:1]
    mean = ssum / jnp.maximum(cnt, 1.0)
    out = (jnp.dot(mean, wl_ref[:], preferred_element_type=jnp.float32)
           + bl_ref[:]
           + jnp.dot(x_ref[:], wr_ref[:], preferred_element_type=jnp.float32))
    nrm = jnp.sqrt(jnp.sum(out * out, axis=1, keepdims=True))
    out_ref[:] = out / jnp.maximum(nrm, 1e-12)


def _tc_dense(psum, pcnt, x, W_l, b_l, W_r):
    return pl.pallas_call(
        _tc_body,
        grid=(N // BN,),
        in_specs=[
            pl.BlockSpec((NC, BN, D), lambda i: (0, i, 0)),
            pl.BlockSpec((NC, BN, D), lambda i: (0, i, 0)),
            pl.BlockSpec((BN, D), lambda i: (i, 0)),
            pl.BlockSpec((D, D), lambda i: (0, 0)),
            pl.BlockSpec((1, D), lambda i: (0, 0)),
            pl.BlockSpec((D, D), lambda i: (0, 0)),
        ],
        out_specs=pl.BlockSpec((BN, D), lambda i: (i, 0)),
        out_shape=jax.ShapeDtypeStruct((N, D), jnp.float32),
    )(psum, pcnt, x, W_l, b_l, W_r)


def kernel(x, edge_index, W_l, b_l, W_r):
    eidx = edge_index.reshape(2, NW, NCHUNKS, CHUNK)
    psum, pcnt = _sc_agg(x, eidx)
    return _tc_dense(psum, pcnt, x, W_l, b_l.reshape(1, D), W_r)
